# Initial kernel scaffold; baseline (speedup 1.0000x reference)
import functools

import jax
import jax.numpy as jnp
from jax import lax
from jax.experimental import pallas as pl
from jax.experimental.pallas import tpu as pltpu
from jax.experimental.pallas import tpu_sc as plsc

H = W = 512
N = H * W
NLOW = int(0.3 * N)
KSAMP = 5000
NT = 16
CH1 = N // NT
M2 = 78848
CH2 = M2 // NT
RAD = 2048
PAD_U = -8388608


def _tc_prep(cam_ref, u_ref, bits_ref, logw_ref, gum_ref):
  x = cam_ref[...] + 1e-8
  bits_ref[...] = lax.bitcast_convert_type(x, jnp.int32)
  logw_ref[...] = jnp.log(jnp.maximum(1.0 - x, 0.0) + 1e-8)
  u = u_ref[...]
  gum_ref[...] = -jnp.log(-jnp.log(u))


def _dyn_gather(x, idx):
  return lax.gather(
      x, idx[:, None],
      dimension_numbers=lax.GatherDimensionNumbers(
          offset_dims=(), collapsed_slice_dims=(0,), start_index_map=(0,)),
      slice_sizes=(1,),
      mode=lax.GatherScatterMode.PROMISE_IN_BOUNDS)


def _sc_body(bits_hbm, logw_hbm, gum_hbm, bg_hbm, out_hbm,
             sAv, sAi, sBv, sBi, sH, sQv, sQi,
             vval, vidx, vpos, vallh, vhist, vnxt,
             vlp, vg, vu2, vp2, vpos2, vones, vp5, vchosen, vtmp16,
             sem, sem2):
  tid = lax.axis_index("s")
  iota = lax.iota(jnp.int32, 16)

  b1 = tid * CH1
  pltpu.sync_copy(bg_hbm.at[pl.ds(b1, CH1)], out_hbm.at[pl.ds(b1, CH1)])

  def place16(d):
    ck = d * 16 + iota
    sk, _ = plsc.sort_key_val(ck, iota)
    sd = lax.shift_right_logical(sk, jnp.full((16,), 4, jnp.int32))
    sl = lax.bitwise_and(sk, jnp.full((16,), 15, jnp.int32))
    prev = _dyn_gather(sd, jnp.maximum(iota - 1, 0))
    nxt_ = _dyn_gather(sd, jnp.minimum(iota + 1, 15))
    rs = jnp.logical_or(iota == 0, sd != prev)
    le = jnp.logical_or(iota == 15, sd != nxt_)
    runbase = plsc.cummax(jnp.where(rs, iota, 0))
    occ = iota - runbase
    cur = plsc.load_gather(vnxt, [sd])
    pos_sorted = cur + occ
    plsc.store_scatter(vnxt, [sd], cur + occ + 1, mask=le)
    plsc.store_scatter(vtmp16, [sl], pos_sorted)
    return vtmp16[...]

  def hist16(d):
    ck = d * 16 + iota
    sk, _ = plsc.sort_key_val(ck, iota)
    sd = lax.shift_right_logical(sk, jnp.full((16,), 4, jnp.int32))
    prev = _dyn_gather(sd, jnp.maximum(iota - 1, 0))
    nxt_ = _dyn_gather(sd, jnp.minimum(iota + 1, 15))
    rs = jnp.logical_or(iota == 0, sd != prev)
    le = jnp.logical_or(iota == 15, sd != nxt_)
    runbase = plsc.cummax(jnp.where(rs, iota, 0))
    occ = iota - runbase
    plsc.addupdate_scatter(vhist, [sd], occ + 1, mask=le)

  def digit_of(v, shift, nbits):
    s = lax.shift_right_logical(v, jnp.full((16,), shift, jnp.int32))
    return lax.bitwise_and(s, jnp.full((16,), (1 << nbits) - 1, jnp.int32))

  def zero_hist():
    def zb(i, _):
      vhist[pl.ds(i * 16, 16)] = jnp.zeros((16,), jnp.int32)
      return 0
    lax.fori_loop(0, RAD // 16, zb, 0)

  def exchange_and_prefix():
    pltpu.sync_copy(vhist, sH.at[pl.ds(tid * RAD, RAD)])
    plsc.subcore_barrier()
    pltpu.sync_copy(sH, vallh)

    def grp(g, carry):
      tot = jnp.zeros((16,), jnp.int32)
      pre = jnp.zeros((16,), jnp.int32)
      for t in range(NT):
        h = vallh[pl.ds(t * RAD + g * 16, 16)]
        pre = pre + jnp.where(jnp.full((16,), t, jnp.int32) < tid, h, 0)
        tot = tot + h
      excl = plsc.cumsum(tot) - tot
      vnxt[pl.ds(g * 16, 16)] = carry + excl + pre
      return carry + lax.reduce_sum(tot, axes=(0,))
    lax.fori_loop(0, RAD // 16, grp, jnp.int32(0))

  def radix_pass(src_v, src_i, dst_v, dst_i, shift, nbits, load_input=True):
    nv = CH1 // 16
    if load_input:
      pltpu.sync_copy(src_v.at[pl.ds(b1, CH1)], vval)
      pltpu.sync_copy(src_i.at[pl.ds(b1, CH1)], vidx)
    zero_hist()

    def hb(i, _):
      hist16(digit_of(vval[pl.ds(i * 16, 16)], shift, nbits))
      return 0
    lax.fori_loop(0, nv, hb, 0)
    exchange_and_prefix()

    def pb(i, _):
      vpos[pl.ds(i * 16, 16)] = place16(
          digit_of(vval[pl.ds(i * 16, 16)], shift, nbits))
      return 0
    lax.fori_loop(0, nv, pb, 0)

    pltpu.make_async_copy(vval, dst_v.at[vpos], sem).start()
    pltpu.make_async_copy(vidx, dst_i.at[vpos], sem2).start()
    pltpu.make_async_copy(vval, dst_v.at[vpos], sem).wait()
    pltpu.make_async_copy(vidx, dst_i.at[vpos], sem2).wait()
    plsc.subcore_barrier()

  pltpu.sync_copy(bits_hbm.at[pl.ds(b1, CH1)], vval)

  def ib(i, _):
    vidx[pl.ds(i * 16, 16)] = b1 + i * 16 + iota
    return 0
  lax.fori_loop(0, CH1 // 16, ib, 0)
  radix_pass(None, None, sAv, sAi, 0, 10, load_input=False)
  radix_pass(sAv, sAi, sBv, sBi, 10, 10)
  radix_pass(sBv, sBi, sAv, sAi, 20, 10)

  b2 = tid * CH2
  pltpu.sync_copy(sAi.at[pl.ds(b2, CH2)], vu2)
  pltpu.make_async_copy(logw_hbm.at[vu2], vlp, sem).start()
  pltpu.make_async_copy(logw_hbm.at[vu2], vlp, sem).wait()
  pltpu.sync_copy(gum_hbm.at[pl.ds(b2, CH2)], vg)

  def kb(i, _):
    key = vlp[pl.ds(i * 16, 16)] + vg[pl.ds(i * 16, 16)]
    kbits = plsc.bitcast(key, jnp.int32)
    s = jnp.where(kbits >= 0, kbits,
                  lax.bitwise_xor(kbits, jnp.full((16,), 0x7FFFFFFF,
                                                  jnp.int32)))
    u = lax.bitwise_xor(jnp.invert(s), jnp.full((16,), -2147483648,
                                                jnp.int32))
    p = b2 + i * 16 + iota
    vu2[pl.ds(i * 16, 16)] = jnp.where(p < NLOW, u,
                                       jnp.full((16,), PAD_U, jnp.int32))
    vp2[pl.ds(i * 16, 16)] = p
    return 0
  lax.fori_loop(0, CH2 // 16, kb, 0)

  def radix_pass2(src_v, src_i, dst_v, dst_i, shift, nbits, load_input=True):
    nv = CH2 // 16
    if load_input:
      pltpu.sync_copy(src_v.at[pl.ds(b2, CH2)], vu2)
      pltpu.sync_copy(src_i.at[pl.ds(b2, CH2)], vp2)
    zero_hist()

    def hb(i, _):
      hist16(digit_of(vu2[pl.ds(i * 16, 16)], shift, nbits))
      return 0
    lax.fori_loop(0, nv, hb, 0)
    exchange_and_prefix()

    def pb(i, _):
      vpos2[pl.ds(i * 16, 16)] = place16(
          digit_of(vu2[pl.ds(i * 16, 16)], shift, nbits))
      return 0
    lax.fori_loop(0, nv, pb, 0)
    pltpu.make_async_copy(vu2, dst_v.at[vpos2], sem).start()
    pltpu.make_async_copy(vp2, dst_i.at[vpos2], sem2).start()
    pltpu.make_async_copy(vu2, dst_v.at[vpos2], sem).wait()
    pltpu.make_async_copy(vp2, dst_i.at[vpos2], sem2).wait()
    plsc.subcore_barrier()

  radix_pass2(None, None, sBv, sBi, 0, 11, load_input=False)
  radix_pass2(sBv, sBi, sQv, sQi, 11, 11)
  radix_pass2(sQv, sQi, sBv, sBi, 22, 10)

  def ob(i, _):
    vones[pl.ds(i * 16, 16)] = jnp.full((16,), 1.0, jnp.float32)
    return 0
  lax.fori_loop(0, KSAMP // 16 + 1, ob, 0)

  @pl.when(tid == 0)
  def _():
    pltpu.sync_copy(sBi.at[pl.ds(0, KSAMP)], vp5)
    pltpu.make_async_copy(sAi.at[vp5], vchosen, sem).start()
    pltpu.make_async_copy(sAi.at[vp5], vchosen, sem).wait()
    pltpu.make_async_copy(vones.at[pl.ds(0, KSAMP)], out_hbm.at[vchosen],
                          sem).start()
    pltpu.make_async_copy(vones.at[pl.ds(0, KSAMP)], out_hbm.at[vchosen],
                          sem).wait()


_sc_kernel = functools.partial(
    pl.kernel,
    out_type=jax.ShapeDtypeStruct((N,), jnp.float32),
    mesh=plsc.VectorSubcoreMesh(
        core_axis_name="c", subcore_axis_name="s", num_cores=1),
    compiler_params=pltpu.CompilerParams(needs_layout_passes=False),
    scratch_types=[
        pltpu.VMEM_SHARED((N,), jnp.int32),
        pltpu.VMEM_SHARED((N,), jnp.int32),
        pltpu.VMEM_SHARED((N,), jnp.int32),
        pltpu.VMEM_SHARED((N,), jnp.int32),
        pltpu.VMEM_SHARED((NT * RAD,), jnp.int32),
        pltpu.VMEM_SHARED((M2,), jnp.int32),
        pltpu.VMEM_SHARED((M2,), jnp.int32),
        pltpu.VMEM((CH1,), jnp.int32),
        pltpu.VMEM((CH1,), jnp.int32),
        pltpu.VMEM((CH1,), jnp.int32),
        pltpu.VMEM((NT * RAD,), jnp.int32),
        pltpu.VMEM((RAD,), jnp.int32),
        pltpu.VMEM((RAD,), jnp.int32),
        pltpu.VMEM((CH2,), jnp.float32),
        pltpu.VMEM((CH2,), jnp.float32),
        pltpu.VMEM((CH2,), jnp.int32),
        pltpu.VMEM((CH2,), jnp.int32),
        pltpu.VMEM((CH2,), jnp.int32),
        pltpu.VMEM((KSAMP + 16,), jnp.float32),
        pltpu.VMEM((KSAMP,), jnp.int32),
        pltpu.VMEM((KSAMP,), jnp.int32),
        pltpu.VMEM((16,), jnp.int32),
        pltpu.SemaphoreType.DMA,
        pltpu.SemaphoreType.DMA,
    ],
)(_sc_body)


def kernel(cam, bg):
  u = jax.random.uniform(jax.random.key(1234), (NLOW,), dtype=jnp.float32,
                         minval=1e-12, maxval=1.0)
  u = jnp.concatenate([u, jnp.full((M2 - NLOW,), 0.5, jnp.float32)])
  bits, logw, gum = pl.pallas_call(
      _tc_prep,
      grid=(4,),
      in_specs=[
          pl.BlockSpec((H // 4, W), lambda i: (i, 0)),
          pl.BlockSpec((M2 // 4,), lambda i: (i,)),
      ],
      out_specs=[
          pl.BlockSpec((H // 4, W), lambda i: (i, 0)),
          pl.BlockSpec((H // 4, W), lambda i: (i, 0)),
          pl.BlockSpec((M2 // 4,), lambda i: (i,)),
      ],
      out_shape=[
          jax.ShapeDtypeStruct((H, W), jnp.int32),
          jax.ShapeDtypeStruct((H, W), jnp.float32),
          jax.ShapeDtypeStruct((M2,), jnp.float32),
      ],
  )(cam, u)
  out = _sc_kernel(bits.reshape(N), logw.reshape(N), gum, bg.reshape(N))
  return out.reshape(H, W)


# speedup vs baseline: 2.3000x; 2.3000x over previous
import functools

import jax
import jax.numpy as jnp
from jax import lax
from jax.experimental import pallas as pl
from jax.experimental.pallas import tpu as pltpu
from jax.experimental.pallas import tpu_sc as plsc

H = W = 512
N = H * W
NLOW = int(0.3 * N)
KSAMP = 5000
NT = 16
CH1 = N // NT
SUB = 4096
M2 = 78848
CH2 = M2 // NT
RAD = 256
PAD_U = -8388608
K3A = 2504
K3B = KSAMP - K3A


def _tc_prep(cam_ref, bits_ref, logw_ref):
  x = cam_ref[...] + 1e-8
  bits_ref[...] = lax.bitcast_convert_type(x, jnp.int32)
  logw_ref[...] = jnp.log(jnp.maximum(1.0 - x, 0.0) + 1e-8)


def _tc_gumbel(u_ref, gum_ref):
  gum_ref[...] = -jnp.log(-jnp.log(u_ref[...]))


def _dyn_gather(x, idx):
  return lax.gather(
      x, idx[:, None],
      dimension_numbers=lax.GatherDimensionNumbers(
          offset_dims=(), collapsed_slice_dims=(0,), start_index_map=(0,)),
      slice_sizes=(1,),
      mode=lax.GatherScatterMode.PROMISE_IN_BOUNDS)


def _sc_body(bits_hbm, logw_hbm, gum_hbm, bg_hbm, out_hbm,
             sAv, sAi, sBv, sBi, sH, sQv, sQi,
             vval, vidx, vpos, vallh, vhist, vnxt,
             vlp, vg, vu2, vp2, vpos2, vch1, vch2, vtmp16,
             sem, sem2):
  tid = lax.axis_index("s")
  iota = lax.iota(jnp.int32, 16)

  b1 = tid * CH1
  pltpu.sync_copy(bg_hbm.at[pl.ds(b1, CH1)], out_hbm.at[pl.ds(b1, CH1)])

  def sort_runs(d):
    ck = d * 16 + iota
    sk, _ = plsc.sort_key_val(ck, iota)
    sd = lax.shift_right_logical(sk, jnp.full((16,), 4, jnp.int32))
    sl = lax.bitwise_and(sk, jnp.full((16,), 15, jnp.int32))
    prev = _dyn_gather(sd, jnp.maximum(iota - 1, 0))
    nxt_ = _dyn_gather(sd, jnp.minimum(iota + 1, 15))
    rs = jnp.logical_or(iota == 0, sd != prev)
    le = jnp.logical_or(iota == 15, sd != nxt_)
    runbase = plsc.cummax(jnp.where(rs, iota, 0))
    occ = iota - runbase
    return sd, sl, occ, le

  def hist16(d):
    sd, _, occ, le = sort_runs(d)
    plsc.addupdate_scatter(vhist, [sd], occ + 1, mask=le)

  def place16(d):
    sd, sl, occ, le = sort_runs(d)
    cur = plsc.load_gather(vnxt, [sd])
    plsc.store_scatter(vnxt, [sd], cur + occ + 1, mask=le)
    plsc.store_scatter(vtmp16, [sl], cur + occ)
    return vtmp16[...]

  def digit_of(v, shift):
    s = lax.shift_right_logical(v, jnp.full((16,), shift, jnp.int32))
    return lax.bitwise_and(s, jnp.full((16,), RAD - 1, jnp.int32))

  def zero_hist():
    def zb(i, _):
      vhist[pl.ds(i * 16, 16)] = jnp.zeros((16,), jnp.int32)
      return 0
    lax.fori_loop(0, RAD // 16, zb, 0)

  def exchange_and_prefix():
    pltpu.sync_copy(vhist, sH.at[pl.ds(tid * RAD, RAD)])
    plsc.subcore_barrier()
    pltpu.sync_copy(sH, vallh)

    def grp(g, carry):
      tot = jnp.zeros((16,), jnp.int32)
      pre = jnp.zeros((16,), jnp.int32)
      for t in range(NT):
        h = vallh[pl.ds(t * RAD + g * 16, 16)]
        pre = pre + jnp.where(jnp.full((16,), t, jnp.int32) < tid, h, 0)
        tot = tot + h
      excl = plsc.cumsum(tot) - tot
      vnxt[pl.ds(g * 16, 16)] = carry + excl + pre
      return carry + jnp.sum(tot)
    lax.fori_loop(0, RAD // 16, grp, jnp.int32(0))

  def hist_scan(shift, nv):
    def hb(i, _):
      hist16(digit_of(vval[pl.ds(i * 16, 16)], shift))
      return 0
    lax.fori_loop(0, nv, hb, 0)

  def place_scan(shift, nv, posbuf):
    def pb(i, _):
      posbuf[pl.ds(i * 16, 16)] = place16(
          digit_of(vval[pl.ds(i * 16, 16)], shift))
      return 0
    lax.fori_loop(0, nv, pb, 0)

  def radix_pass1(src_v, src_i, dst_v, dst_i, shift, first=False,
                  write_val=True):
    zero_hist()
    for s in range(CH1 // SUB):
      off = b1 + s * SUB
      pltpu.sync_copy((bits_hbm if first else src_v).at[pl.ds(off, SUB)],
                      vval)
      hist_scan(shift, SUB // 16)
    exchange_and_prefix()
    for s in range(CH1 // SUB):
      off = b1 + s * SUB
      pltpu.sync_copy((bits_hbm if first else src_v).at[pl.ds(off, SUB)],
                      vval)
      if first:
        def ib(i, _):
          vidx[pl.ds(i * 16, 16)] = off + i * 16 + iota
          return 0
        lax.fori_loop(0, SUB // 16, ib, 0)
      else:
        pltpu.sync_copy(src_i.at[pl.ds(off, SUB)], vidx)
      place_scan(shift, SUB // 16, vpos)
      if write_val:
        pltpu.make_async_copy(vval, dst_v.at[vpos], sem).start()
      pltpu.make_async_copy(vidx, dst_i.at[vpos], sem2).start()
      if write_val:
        pltpu.make_async_copy(vval, dst_v.at[vpos], sem).wait()
      pltpu.make_async_copy(vidx, dst_i.at[vpos], sem2).wait()
    plsc.subcore_barrier()

  radix_pass1(None, None, sAv, sAi, 0, first=True)
  radix_pass1(sAv, sAi, sBv, sBi, 8)
  radix_pass1(sBv, sBi, sAv, sAi, 16)
  radix_pass1(sAv, sAi, sBv, sBi, 24, write_val=False)

  b2 = tid * CH2
  pltpu.sync_copy(sBi.at[pl.ds(b2, CH2)], vu2)
  pltpu.make_async_copy(logw_hbm.at[vu2], vlp, sem).start()
  pltpu.make_async_copy(logw_hbm.at[vu2], vlp, sem).wait()
  pltpu.sync_copy(gum_hbm.at[pl.ds(b2, CH2)], vg)

  def kb(i, _):
    key = vlp[pl.ds(i * 16, 16)] + vg[pl.ds(i * 16, 16)]
    kbits = plsc.bitcast(key, jnp.int32)
    s = jnp.where(kbits >= 0, kbits,
                  lax.bitwise_xor(kbits, jnp.full((16,), 0x7FFFFFFF,
                                                  jnp.int32)))
    u = lax.bitwise_xor(jnp.invert(s), jnp.full((16,), -2147483648,
                                                jnp.int32))
    p = b2 + i * 16 + iota
    vu2[pl.ds(i * 16, 16)] = jnp.where(p < NLOW, u,
                                       jnp.full((16,), PAD_U, jnp.int32))
    vp2[pl.ds(i * 16, 16)] = p
    return 0
  lax.fori_loop(0, CH2 // 16, kb, 0)

  def radix_pass2(src_v, src_i, dst_v, dst_i, shift, write_val=True):
    zero_hist()
    if src_v is not None:
      pltpu.sync_copy(src_v.at[pl.ds(b2, CH2)], vu2)
      pltpu.sync_copy(src_i.at[pl.ds(b2, CH2)], vp2)

    def hb(i, _):
      hist16(digit_of(vu2[pl.ds(i * 16, 16)], shift))
      return 0
    lax.fori_loop(0, CH2 // 16, hb, 0)
    exchange_and_prefix()

    def pb(i, _):
      vpos2[pl.ds(i * 16, 16)] = place16(
          digit_of(vu2[pl.ds(i * 16, 16)], shift))
      return 0
    lax.fori_loop(0, CH2 // 16, pb, 0)
    if write_val:
      pltpu.make_async_copy(vu2, dst_v.at[vpos2], sem).start()
    pltpu.make_async_copy(vp2, dst_i.at[vpos2], sem2).start()
    if write_val:
      pltpu.make_async_copy(vu2, dst_v.at[vpos2], sem).wait()
    pltpu.make_async_copy(vp2, dst_i.at[vpos2], sem2).wait()
    plsc.subcore_barrier()

  radix_pass2(None, None, sQv, sQi, 0)
  radix_pass2(sQv, sQi, sAv, sAi, 8)
  radix_pass2(sAv, sAi, sQv, sQi, 16)
  radix_pass2(sQv, sQi, sAv, sAi, 24, write_val=False)

  def ob(i, _):
    vg[pl.ds(i * 16, 16)] = jnp.full((16,), 1.0, jnp.float32)
    return 0
  lax.fori_loop(0, K3A // 16 + 1, ob, 0)

  @pl.when(tid == 0)
  def _():
    pltpu.sync_copy(sAi.at[pl.ds(0, K3A)], vp2.at[pl.ds(0, K3A)])
    pltpu.make_async_copy(sBi.at[vp2.at[pl.ds(0, K3A)]], vch1, sem).start()
    pltpu.make_async_copy(sBi.at[vp2.at[pl.ds(0, K3A)]], vch1, sem).wait()
    pltpu.make_async_copy(vg.at[pl.ds(0, K3A)], out_hbm.at[vch1],
                          sem).start()
    pltpu.make_async_copy(vg.at[pl.ds(0, K3A)], out_hbm.at[vch1],
                          sem).wait()
    pltpu.sync_copy(sAi.at[pl.ds(K3A, K3B)], vp2.at[pl.ds(0, K3B)])
    pltpu.make_async_copy(sBi.at[vp2.at[pl.ds(0, K3B)]], vch2, sem).start()
    pltpu.make_async_copy(sBi.at[vp2.at[pl.ds(0, K3B)]], vch2, sem).wait()
    pltpu.make_async_copy(vg.at[pl.ds(0, K3B)], out_hbm.at[vch2],
                          sem).start()
    pltpu.make_async_copy(vg.at[pl.ds(0, K3B)], out_hbm.at[vch2],
                          sem).wait()


_sc_kernel = functools.partial(
    pl.kernel,
    out_type=jax.ShapeDtypeStruct((N,), jnp.float32),
    mesh=plsc.VectorSubcoreMesh(
        core_axis_name="c", subcore_axis_name="s", num_cores=1),
    compiler_params=pltpu.CompilerParams(needs_layout_passes=False),
    scratch_types=[
        pltpu.VMEM_SHARED((N,), jnp.int32),
        pltpu.VMEM_SHARED((N,), jnp.int32),
        pltpu.VMEM_SHARED((N,), jnp.int32),
        pltpu.VMEM_SHARED((N,), jnp.int32),
        pltpu.VMEM_SHARED((NT * RAD,), jnp.int32),
        pltpu.VMEM_SHARED((M2,), jnp.int32),
        pltpu.VMEM_SHARED((M2,), jnp.int32),
        pltpu.VMEM((SUB,), jnp.int32),
        pltpu.VMEM((SUB,), jnp.int32),
        pltpu.VMEM((SUB,), jnp.int32),
        pltpu.VMEM((NT * RAD,), jnp.int32),
        pltpu.VMEM((RAD,), jnp.int32),
        pltpu.VMEM((RAD,), jnp.int32),
        pltpu.VMEM((CH2,), jnp.float32),
        pltpu.VMEM((CH2,), jnp.float32),
        pltpu.VMEM((CH2,), jnp.int32),
        pltpu.VMEM((CH2,), jnp.int32),
        pltpu.VMEM((CH2,), jnp.int32),
        pltpu.VMEM((K3A,), jnp.int32),
        pltpu.VMEM((K3B,), jnp.int32),
        pltpu.VMEM((16,), jnp.int32),
        pltpu.SemaphoreType.DMA,
        pltpu.SemaphoreType.DMA,
    ],
)(_sc_body)


def kernel(cam, bg):
  u = jax.random.uniform(jax.random.key(1234), (NLOW,), dtype=jnp.float32,
                         minval=1e-12, maxval=1.0)
  u = jnp.concatenate([u, jnp.full((M2 - NLOW,), 0.5, jnp.float32)])
  u = u.reshape(M2 // 128, 128)
  bits, logw = pl.pallas_call(
      _tc_prep,
      grid=(4,),
      in_specs=[pl.BlockSpec((H // 4, W), lambda i: (i, 0))],
      out_specs=[
          pl.BlockSpec((H // 4, W), lambda i: (i, 0)),
          pl.BlockSpec((H // 4, W), lambda i: (i, 0)),
      ],
      out_shape=[
          jax.ShapeDtypeStruct((H, W), jnp.int32),
          jax.ShapeDtypeStruct((H, W), jnp.float32),
      ],
  )(cam)
  gum = pl.pallas_call(
      _tc_gumbel,
      out_shape=jax.ShapeDtypeStruct((M2 // 128, 128), jnp.float32),
  )(u)
  out = _sc_kernel(bits.reshape(N), logw.reshape(N), gum.reshape(M2),
                   bg.reshape(N))
  return out.reshape(H, W)
